# baseline (device time: 414308 ns/iter reference)
import jax
import jax.numpy as jnp
from jax import lax
from jax.experimental import pallas as pl
from jax.experimental.pallas import tpu as pltpu

N_DEV = 32
H_R = 16
H_L = 15
SUB_R = 2 * H_R
SUB_L = 2 * H_L
S = 6

_Q = {(0, 0): 0, (1, 0): 1, (1, 1): 2, (0, 1): 3,
      (0, 2): 4, (1, 2): 5, (1, 3): 6, (0, 3): 7}


def _logical_id(x: int, y: int, z: int) -> int:
    return z * 8 + _Q[(x, y)]


_P0 = [(y, 0) for y in range(4)] + [(y, 1) for y in reversed(range(4))] + \
      [(y, 2) for y in range(4)] + [(y, 3) for y in reversed(range(4))]
_COORD_CYCLE = [(0, y, z) for (y, z) in _P0] + \
               [(1, y, z) for (y, z) in reversed(_P0)]
for _a, _b in zip(_COORD_CYCLE, _COORD_CYCLE[1:] + _COORD_CYCLE[:1]):
    assert sum(abs(i - j) for i, j in zip(_a, _b)) == 1, (_a, _b)

RING = [_logical_id(*c) for c in _COORD_CYCLE]
POS = [RING.index(i) for i in range(N_DEV)]
assert sorted(RING) == list(range(N_DEV))


def kernel(x):
    m_per, n = x.shape
    m_sub = m_per // 2

    def body(x_ref, pos_ref, ring_ref, out_ref,
             vmem_r, vmem_l, send_r, recv_r, send_l, recv_l,
             lc_r, lc_l, credit_r, credit_l, local_sem):
        me = lax.axis_index("i")
        p = pos_ref[me]
        right = ring_ref[lax.rem(p + 1, N_DEV)]
        left = ring_ref[lax.rem(p + N_DEV - 1, N_DEV)]

        barrier = pltpu.get_barrier_semaphore()
        for nbr in (left, right):
            pl.semaphore_signal(
                barrier, inc=1,
                device_id=(nbr,), device_id_type=pl.DeviceIdType.MESH,
            )
        pl.semaphore_wait(barrier, 2)

        own = pltpu.make_async_copy(
            x_ref, out_ref.at[pl.ds(me * m_per, m_per), :], local_sem
        )
        own.start()

        def send_desc(t, rightward):
            vmem = vmem_r if rightward else vmem_l
            sems = (send_r, recv_r) if rightward else (send_l, recv_l)
            tgt = right if rightward else left
            if t < 2:
                src = x_ref.at[pl.ds((t % 2) * m_sub, m_sub), :]
            else:
                src = vmem.at[(t - 2) % S]
            return pltpu.make_async_remote_copy(
                src_ref=src, dst_ref=vmem.at[t % S],
                send_sem=sems[0].at[t % S], recv_sem=sems[1].at[t % S],
                device_id=(tgt,), device_id_type=pl.DeviceIdType.MESH,
            )

        def recv_desc(t, rightward):
            vmem = vmem_r if rightward else vmem_l
            sems = (send_r, recv_r) if rightward else (send_l, recv_l)
            frm = left if rightward else right
            return pltpu.make_async_remote_copy(
                src_ref=vmem.at[t % S], dst_ref=vmem.at[t % S],
                send_sem=sems[0].at[t % S], recv_sem=sems[1].at[t % S],
                device_id=(frm,), device_id_type=pl.DeviceIdType.MESH,
            )

        def drain_desc(t, rightward):
            k, half = t // 2, t % 2
            if rightward:
                chunk = ring_ref[lax.rem(p - 1 - k + N_DEV, N_DEV)]
                vmem, lc = vmem_r, lc_r
            else:
                chunk = ring_ref[lax.rem(p + 1 + k, N_DEV)]
                vmem, lc = vmem_l, lc_l
            dst = out_ref.at[
                (pl.ds(chunk * m_per + half * m_sub, m_sub), slice(None))
            ]
            return pltpu.make_async_copy(vmem.at[t % S], dst, lc.at[t % S])

        sends_r = {}
        sends_l = {}
        for t in (0, 1):
            for rightward in (True, False):
                d = send_desc(t, rightward)
                d.start()
                (sends_r if rightward else sends_l)[t] = d

        for t in range(2, SUB_R):
            recv_desc(t - 2, True).wait_recv()
            drain_desc(t - 2, True).start()
            if t >= S:
                pl.semaphore_wait(credit_r, 1)
            d = send_desc(t, True)
            d.start()
            sends_r[t] = d
            sends_r[t - 2].wait_send()
            if t >= 4:
                drain_desc(t - 4, True).wait()
            if 4 <= t <= 29:
                pl.semaphore_signal(
                    credit_r, inc=1,
                    device_id=(left,), device_id_type=pl.DeviceIdType.MESH,
                )
            if t < SUB_L:
                recv_desc(t - 2, False).wait_recv()
                drain_desc(t - 2, False).start()
                if t >= S:
                    pl.semaphore_wait(credit_l, 1)
                d = send_desc(t, False)
                d.start()
                sends_l[t] = d
                sends_l[t - 2].wait_send()
                if t >= 4:
                    drain_desc(t - 4, False).wait()
                if 4 <= t <= 27:
                    pl.semaphore_signal(
                        credit_l, inc=1,
                        device_id=(right,), device_id_type=pl.DeviceIdType.MESH,
                    )

        for t in (SUB_R - 2, SUB_R - 1):
            recv_desc(t, True).wait_recv()
            drain_desc(t, True).start()
        for t in (SUB_L - 2, SUB_L - 1):
            recv_desc(t, False).wait_recv()
            drain_desc(t, False).start()

        for t in (SUB_R - 2, SUB_R - 1):
            sends_r[t].wait_send()
        for t in (SUB_L - 2, SUB_L - 1):
            sends_l[t].wait_send()
        for t in range(SUB_R - 4, SUB_R):
            drain_desc(t, True).wait()
        for t in range(SUB_L - 4, SUB_L):
            drain_desc(t, False).wait()
        own.wait()

    pos_tab = jnp.asarray(POS, dtype=jnp.int32)
    ring_tab = jnp.asarray(RING, dtype=jnp.int32)

    return pl.pallas_call(
        body,
        out_shape=jax.ShapeDtypeStruct((N_DEV * m_per, n), x.dtype),
        in_specs=[
            pl.BlockSpec(memory_space=pltpu.VMEM),
            pl.BlockSpec(memory_space=pltpu.SMEM),
            pl.BlockSpec(memory_space=pltpu.SMEM),
        ],
        out_specs=pl.BlockSpec(memory_space=pl.ANY),
        scratch_shapes=[
            pltpu.VMEM((S, m_sub, n), x.dtype),
            pltpu.VMEM((S, m_sub, n), x.dtype),
            pltpu.SemaphoreType.DMA((S,)),
            pltpu.SemaphoreType.DMA((S,)),
            pltpu.SemaphoreType.DMA((S,)),
            pltpu.SemaphoreType.DMA((S,)),
            pltpu.SemaphoreType.DMA((S,)),
            pltpu.SemaphoreType.DMA((S,)),
            pltpu.SemaphoreType.REGULAR,
            pltpu.SemaphoreType.REGULAR,
            pltpu.SemaphoreType.DMA,
        ],
        compiler_params=pltpu.CompilerParams(collective_id=0),
    )(x, pos_tab, ring_tab)


# device time: 360061 ns/iter; 1.1507x vs baseline; 1.1507x over previous
import jax
import jax.numpy as jnp
from jax import lax
from jax.experimental import pallas as pl
from jax.experimental.pallas import tpu as pltpu

N_DEV = 32
D = 13
SUB = 2 * D
N_RELAY = N_DEV - 1 - 2 * D

_Q = {(0, 0): 0, (1, 0): 1, (1, 1): 2, (0, 1): 3,
      (0, 2): 4, (1, 2): 5, (1, 3): 6, (0, 3): 7}


def _logical_id(x: int, y: int, z: int) -> int:
    return z * 8 + _Q[(x, y)]


_P0 = [(y, 0) for y in range(4)] + [(y, 1) for y in reversed(range(4))] + \
      [(y, 2) for y in range(4)] + [(y, 3) for y in reversed(range(4))]
_COORD_CYCLE = [(0, y, z) for (y, z) in _P0] + \
               [(1, y, z) for (y, z) in reversed(_P0)]
for _a, _b in zip(_COORD_CYCLE, _COORD_CYCLE[1:] + _COORD_CYCLE[:1]):
    assert sum(abs(i - j) for i, j in zip(_a, _b)) == 1, (_a, _b)

RING = [_logical_id(*c) for c in _COORD_CYCLE]
POS = [RING.index(i) for i in range(N_DEV)]
assert sorted(RING) == list(range(N_DEV))

_PAIRS = [(0, 7), (3, 28), (12, 19), (15, 8), (31, 24), (16, 23),
          (4, 27), (5, 26), (6, 1), (2, 29), (25, 30), (9, 14),
          (10, 21), (22, 17), (13, 18), (11, 20)]
PARTNER_POS = [-1] * N_DEV
for _a, _b in _PAIRS:
    assert sum(abs(i - j) for i, j in
               zip(_COORD_CYCLE[_a], _COORD_CYCLE[_b])) == 1, (_a, _b)
    assert (_b - _a) % N_DEV not in (1, N_DEV - 1), (_a, _b)
    _d = (_b - _a) % N_DEV
    assert 31 - 2 * D <= _d <= 2 * D + 1, (_a, _b, _d)
    PARTNER_POS[_a], PARTNER_POS[_b] = _b, _a
assert all(q >= 0 for q in PARTNER_POS)
PARTNER = [0] * N_DEV
for _pp in range(N_DEV):
    PARTNER[RING[_pp]] = RING[PARTNER_POS[_pp]]


def kernel(x):
    m_per, n = x.shape
    m_sub = m_per // 2

    def body(x_ref, pos_ref, ring_ref, partner_ref, out_ref,
             send_r, recv_r, send_l, recv_l,
             relay_send, relay_recv, local_sem):
        me = lax.axis_index("i")
        p = pos_ref[me]
        right = ring_ref[lax.rem(p + 1, N_DEV)]
        left = ring_ref[lax.rem(p + N_DEV - 1, N_DEV)]
        partner = partner_ref[me]
        q_pos = pos_ref[partner]

        barrier = pltpu.get_barrier_semaphore()
        for nbr in (left, right, partner):
            pl.semaphore_signal(
                barrier, inc=1,
                device_id=(nbr,), device_id_type=pl.DeviceIdType.MESH,
            )
        pl.semaphore_wait(barrier, 3)

        own = pltpu.make_async_copy(
            x_ref, out_ref.at[pl.ds(me * m_per, m_per), :], local_sem
        )
        own.start()

        def sub_slice(chunk, half):
            return (pl.ds(chunk * m_per + half * m_sub, m_sub), slice(None))

        def send_desc(t, rightward):
            k, half = t // 2, t % 2
            if rightward:
                chunk = ring_ref[lax.rem(p - k + N_DEV, N_DEV)]
                sems, tgt = (send_r, recv_r), right
            else:
                chunk = ring_ref[lax.rem(p + k, N_DEV)]
                sems, tgt = (send_l, recv_l), left
            dst = out_ref.at[sub_slice(chunk, half)]
            if t < 2:
                src = x_ref.at[pl.ds(half * m_sub, m_sub), :]
            else:
                src = out_ref.at[sub_slice(chunk, half)]
            return pltpu.make_async_remote_copy(
                src_ref=src, dst_ref=dst,
                send_sem=sems[0].at[t], recv_sem=sems[1].at[t],
                device_id=(tgt,), device_id_type=pl.DeviceIdType.MESH,
            )

        def recv_origin(t, rightward):
            k = t // 2
            if rightward:
                return lax.rem(p - 1 - k + N_DEV, N_DEV)
            return lax.rem(p + 1 + k, N_DEV)

        def recv_desc(t, rightward):
            chunk = ring_ref[recv_origin(t, rightward)]
            sems = (send_r, recv_r) if rightward else (send_l, recv_l)
            frm = left if rightward else right
            sl = out_ref.at[sub_slice(chunk, t % 2)]
            return pltpu.make_async_remote_copy(
                src_ref=sl, dst_ref=sl,
                send_sem=sems[0].at[t], recv_sem=sems[1].at[t],
                device_id=(frm,), device_id_type=pl.DeviceIdType.MESH,
            )

        def maybe_relay(t, rightward):
            o = recv_origin(t, rightward)
            in_arc = lax.rem(o - q_pos + 2 * N_DEV - (D + 1), N_DEV) < N_RELAY

            @pl.when(in_arc)
            def _():
                sl = out_ref.at[sub_slice(ring_ref[o], t % 2)]
                pltpu.make_async_remote_copy(
                    src_ref=sl, dst_ref=sl,
                    send_sem=relay_send, recv_sem=relay_recv,
                    device_id=(partner,),
                    device_id_type=pl.DeviceIdType.MESH,
                ).start()

        in_flight = []
        for t in (0, 1):
            for rightward in (True, False):
                d = send_desc(t, rightward)
                d.start()
                in_flight.append(d)
        for t in range(2, SUB):
            for rightward in (True, False):
                recv_desc(t - 2, rightward).wait_recv()
                maybe_relay(t - 2, rightward)
                d = send_desc(t, rightward)
                d.start()
                in_flight.append(d)
        for t in (SUB - 2, SUB - 1):
            for rightward in (True, False):
                recv_desc(t, rightward).wait_recv()
                maybe_relay(t, rightward)

        for d in in_flight:
            d.wait_send()
        own_half = out_ref.at[pl.ds(me * m_per, m_sub), :]
        relay_drain = pltpu.make_async_remote_copy(
            src_ref=own_half, dst_ref=own_half,
            send_sem=relay_send, recv_sem=relay_recv,
            device_id=(partner,), device_id_type=pl.DeviceIdType.MESH,
        )
        for _ in range(2 * N_RELAY):
            relay_drain.wait_send()
        for _ in range(2 * N_RELAY):
            relay_drain.wait_recv()
        own.wait()

    pos_tab = jnp.asarray(POS, dtype=jnp.int32)
    ring_tab = jnp.asarray(RING, dtype=jnp.int32)
    partner_tab = jnp.asarray(PARTNER, dtype=jnp.int32)

    return pl.pallas_call(
        body,
        out_shape=jax.ShapeDtypeStruct((N_DEV * m_per, n), x.dtype),
        in_specs=[
            pl.BlockSpec(memory_space=pltpu.VMEM),
            pl.BlockSpec(memory_space=pltpu.SMEM),
            pl.BlockSpec(memory_space=pltpu.SMEM),
            pl.BlockSpec(memory_space=pltpu.SMEM),
        ],
        out_specs=pl.BlockSpec(memory_space=pl.ANY),
        scratch_shapes=[
            pltpu.SemaphoreType.DMA((SUB,)),
            pltpu.SemaphoreType.DMA((SUB,)),
            pltpu.SemaphoreType.DMA((SUB,)),
            pltpu.SemaphoreType.DMA((SUB,)),
            pltpu.SemaphoreType.DMA,
            pltpu.SemaphoreType.DMA,
            pltpu.SemaphoreType.DMA,
        ],
        compiler_params=pltpu.CompilerParams(collective_id=0),
    )(x, pos_tab, ring_tab, partner_tab)
